# Initial kernel scaffold; baseline (speedup 1.0000x reference)
import jax
import jax.numpy as jnp
from jax.experimental import pallas as pl


def kernel(x, enc_w1, enc_b1, enc_w2, enc_b2, dec_w1, dec_b1, dec_w2, dec_b2, codebook):
    raise NotImplementedError("write your pallas kernel here")


# speedup vs baseline: 1.6504x; 1.6504x over previous
import functools

import jax
import jax.numpy as jnp
from jax import lax
from jax.experimental import pallas as pl
from jax.experimental.pallas import tpu as pltpu
from jax.experimental.pallas import tpu_sc as plsc

LVL = 4
NF = 768
HID = 512
VQ = 256
CB = 8192
B = 8192
BT = 512
NB = B // BT
TOTB = LVL * B


def _enc_argmin_body(x_ref, w1_ref, b1_ref, w2_ref, b2_ref, cb_ref,
                     h_ref, idx_ref, cnorm_ref):
    lvl = pl.program_id(0)
    bt = pl.program_id(1)
    C = cb_ref[0]

    @pl.when(bt == 0)
    def _():
        C2 = C * C
        cnorm_ref[...] = lax.dot_general(
            jnp.ones((8, VQ), jnp.float32), C2,
            (((1,), (1,)), ((), ())),
            preferred_element_type=jnp.float32)

    xi = x_ref[...]
    h1 = jnp.maximum(
        jnp.dot(xi, w1_ref[0], preferred_element_type=jnp.float32) + b1_ref[0],
        0.0)
    h = jnp.dot(h1, w2_ref[0], preferred_element_type=jnp.float32) + b2_ref[0]
    h_ref[0] = h

    s = lax.dot_general(h, C, (((1,), (1,)), ((), ())),
                        preferred_element_type=jnp.float32)
    s = s - 0.5 * cnorm_ref[0:1]
    idx = jnp.argmax(s, axis=1).astype(jnp.int32)
    idx_ref[0] = (idx + lvl * CB).reshape(1, BT)


def _enc_argmin(x, enc_w1, enc_b1, enc_w2, enc_b2, codebook):
    return pl.pallas_call(
        _enc_argmin_body,
        grid=(LVL, NB),
        in_specs=[
            pl.BlockSpec((BT, NF), lambda l, b: (b, l)),
            pl.BlockSpec((1, NF, HID), lambda l, b: (l, 0, 0)),
            pl.BlockSpec((1, 1, HID), lambda l, b: (l, 0, 0)),
            pl.BlockSpec((1, HID, VQ), lambda l, b: (l, 0, 0)),
            pl.BlockSpec((1, 1, VQ), lambda l, b: (l, 0, 0)),
            pl.BlockSpec((1, CB, VQ), lambda l, b: (l, 0, 0)),
        ],
        out_specs=[
            pl.BlockSpec((1, BT, VQ), lambda l, b: (l, b, 0)),
            pl.BlockSpec((1, 1, BT), lambda l, b: (l, 0, b)),
        ],
        out_shape=[
            jax.ShapeDtypeStruct((LVL, B, VQ), jnp.float32),
            jax.ShapeDtypeStruct((LVL, 1, B), jnp.int32),
        ],
        scratch_shapes=[pltpu.VMEM((8, CB), jnp.float32)],
    )(x, enc_w1, enc_b1, enc_w2, enc_b2, codebook)


_SC_CH = 128


def _make_sc_gather():
    info = plsc.get_sparse_core_info()
    nc, ns = info.num_cores, info.num_subcores
    nw = nc * ns
    b_per_w = TOTB // nw
    n_ch = b_per_w // _SC_CH
    mesh = plsc.VectorSubcoreMesh(core_axis_name="c", subcore_axis_name="s")

    @functools.partial(
        pl.kernel,
        mesh=mesh,
        out_type=jax.ShapeDtypeStruct((TOTB, VQ), jnp.float32),
        scratch_types=[
            pltpu.VMEM((_SC_CH,), jnp.int32),
            pltpu.VMEM((_SC_CH, VQ), jnp.float32),
            pltpu.SemaphoreType.DMA,
        ],
    )
    def sc_gather(table_hbm, idx_hbm, out_hbm, idx_v, rows_v, sem):
        wid = lax.axis_index("s") * nc + lax.axis_index("c")
        for c in range(n_ch):
            base = wid * b_per_w + c * _SC_CH
            pltpu.sync_copy(idx_hbm.at[pl.ds(base, _SC_CH)], idx_v)
            pltpu.async_copy(table_hbm.at[idx_v], rows_v, sem).wait()
            pltpu.sync_copy(rows_v, out_hbm.at[pl.ds(base, _SC_CH)])

    return sc_gather


_sc_gather_cache = []


def _sc_gather(table, idx):
    if not _sc_gather_cache:
        _sc_gather_cache.append(_make_sc_gather())
    return _sc_gather_cache[0](table, idx)


def _dec_loss_body(q_ref, h_ref, x_ref, w1_ref, b1_ref, w2_ref, b2_ref,
                   mse_ref, com_ref):
    lvl = pl.program_id(0)
    bt = pl.program_id(1)

    @pl.when((lvl == 0) & (bt == 0))
    def _():
        mse_ref[...] = jnp.zeros((1, 1), jnp.float32)
        com_ref[...] = jnp.zeros((1, 1), jnp.float32)

    q = q_ref[0]
    d1 = jnp.maximum(
        jnp.dot(q, w1_ref[0], preferred_element_type=jnp.float32) + b1_ref[0],
        0.0)
    xh = jnp.dot(d1, w2_ref[0], preferred_element_type=jnp.float32) + b2_ref[0]
    diff = xh - x_ref[...]
    mse_ref[...] += jnp.sum(diff * diff) * (1.0 / (B * NF * LVL))

    @pl.when(lvl == LVL - 1)
    def _():
        dq = q - h_ref[0]
        com_ref[...] += jnp.sum(dq * dq) * (1.0 / (B * VQ * LVL))


def _dec_loss(q, h, x, dec_w1, dec_b1, dec_w2, dec_b2):
    return pl.pallas_call(
        _dec_loss_body,
        grid=(LVL, NB),
        in_specs=[
            pl.BlockSpec((1, BT, VQ), lambda l, b: (l, b, 0)),
            pl.BlockSpec((1, BT, VQ), lambda l, b: (l, b, 0)),
            pl.BlockSpec((BT, NF), lambda l, b: (b, l)),
            pl.BlockSpec((1, VQ, HID), lambda l, b: (l, 0, 0)),
            pl.BlockSpec((1, 1, HID), lambda l, b: (l, 0, 0)),
            pl.BlockSpec((1, HID, NF), lambda l, b: (l, 0, 0)),
            pl.BlockSpec((1, 1, NF), lambda l, b: (l, 0, 0)),
        ],
        out_specs=[
            pl.BlockSpec((1, 1), lambda l, b: (0, 0)),
            pl.BlockSpec((1, 1), lambda l, b: (0, 0)),
        ],
        out_shape=[
            jax.ShapeDtypeStruct((1, 1), jnp.float32),
            jax.ShapeDtypeStruct((1, 1), jnp.float32),
        ],
    )(q, h, x, dec_w1, dec_b1, dec_w2, dec_b2)


def kernel(x, enc_w1, enc_b1, enc_w2, enc_b2, dec_w1, dec_b1, dec_w2, dec_b2,
           codebook):
    eb1 = enc_b1.reshape(LVL, 1, HID)
    eb2 = enc_b2.reshape(LVL, 1, VQ)
    db1 = dec_b1.reshape(LVL, 1, HID)
    db2 = dec_b2.reshape(LVL, 1, NF)

    h, idx = _enc_argmin(x, enc_w1, eb1, enc_w2, eb2, codebook)
    q = _sc_gather(codebook.reshape(LVL * CB, VQ), idx.reshape(TOTB))
    mse, com = _dec_loss(q.reshape(LVL, B, VQ), h, x, dec_w1, db1, dec_w2, db2)
    return (mse.reshape(()), com.reshape(()))


# speedup vs baseline: 1.6606x; 1.0062x over previous
import functools

import jax
import jax.numpy as jnp
from jax import lax
from jax.experimental import pallas as pl
from jax.experimental.pallas import tpu as pltpu
from jax.experimental.pallas import tpu_sc as plsc

LVL = 4
NF = 768
HID = 512
VQ = 256
CB = 8192
B = 8192
BT = 512
NB = B // BT
TOTB = LVL * B


def _enc_argmin_body(x_ref, w1_ref, b1_ref, w2_ref, b2_ref, cb_ref,
                     h_ref, idx_ref, cnorm_ref):
    lvl = pl.program_id(0)
    bt = pl.program_id(1)
    C = cb_ref[0]

    @pl.when(bt == 0)
    def _():
        C2 = C * C
        cnorm_ref[...] = lax.dot_general(
            jnp.ones((8, VQ), jnp.float32), C2,
            (((1,), (1,)), ((), ())),
            preferred_element_type=jnp.float32)

    xi = x_ref[...]
    h1 = jnp.maximum(
        jnp.dot(xi, w1_ref[0], preferred_element_type=jnp.float32) + b1_ref[0],
        0.0)
    h = jnp.dot(h1, w2_ref[0], preferred_element_type=jnp.float32) + b2_ref[0]
    h_ref[0] = h

    s = lax.dot_general(h, C, (((1,), (1,)), ((), ())),
                        preferred_element_type=jnp.float32)
    s = s - 0.5 * cnorm_ref[0:1]
    idx = jnp.argmax(s, axis=1).astype(jnp.int32)
    idx_ref[0] = (idx + lvl * CB).reshape(1, BT)


def _enc_argmin(x, enc_w1, enc_b1, enc_w2, enc_b2, codebook):
    return pl.pallas_call(
        _enc_argmin_body,
        grid=(LVL, NB),
        in_specs=[
            pl.BlockSpec((BT, NF), lambda l, b: (b, l)),
            pl.BlockSpec((1, NF, HID), lambda l, b: (l, 0, 0)),
            pl.BlockSpec((1, 1, HID), lambda l, b: (l, 0, 0)),
            pl.BlockSpec((1, HID, VQ), lambda l, b: (l, 0, 0)),
            pl.BlockSpec((1, 1, VQ), lambda l, b: (l, 0, 0)),
            pl.BlockSpec((1, CB, VQ), lambda l, b: (l, 0, 0)),
        ],
        out_specs=[
            pl.BlockSpec((1, BT, VQ), lambda l, b: (l, b, 0)),
            pl.BlockSpec((1, 1, BT), lambda l, b: (l, 0, b)),
        ],
        out_shape=[
            jax.ShapeDtypeStruct((LVL, B, VQ), jnp.float32),
            jax.ShapeDtypeStruct((LVL, 1, B), jnp.int32),
        ],
        scratch_shapes=[pltpu.VMEM((8, CB), jnp.float32)],
    )(x, enc_w1, enc_b1, enc_w2, enc_b2, codebook)


_SC_CH = 128


def _make_sc_gather():
    info = plsc.get_sparse_core_info()
    nc, ns = info.num_cores, info.num_subcores
    nw = nc * ns
    b_per_w = TOTB // nw
    n_ch = b_per_w // _SC_CH
    mesh = plsc.VectorSubcoreMesh(core_axis_name="c", subcore_axis_name="s")

    @functools.partial(
        pl.kernel,
        mesh=mesh,
        out_type=jax.ShapeDtypeStruct((TOTB, VQ), jnp.float32),
        scratch_types=[
            pltpu.VMEM((b_per_w,), jnp.int32),
            pltpu.VMEM((_SC_CH, VQ), jnp.float32),
            pltpu.VMEM((_SC_CH, VQ), jnp.float32),
            pltpu.SemaphoreType.DMA,
            pltpu.SemaphoreType.DMA,
            pltpu.SemaphoreType.DMA,
        ],
    )
    def sc_gather(table_hbm, idx_hbm, out_hbm, idx_v, buf0, buf1, gsem,
                  osem0, osem1):
        wid = lax.axis_index("s") * nc + lax.axis_index("c")
        base = wid * b_per_w
        bufs = (buf0, buf1)
        osems = (osem0, osem1)
        pltpu.sync_copy(idx_hbm.at[pl.ds(base, b_per_w)], idx_v)
        outcp = [None, None]
        for c in range(n_ch):
            sel = c % 2
            if outcp[sel] is not None:
                outcp[sel].wait()
            pltpu.async_copy(
                table_hbm.at[idx_v.at[pl.ds(c * _SC_CH, _SC_CH)]],
                bufs[sel], gsem).wait()
            outcp[sel] = pltpu.async_copy(
                bufs[sel], out_hbm.at[pl.ds(base + c * _SC_CH, _SC_CH)],
                osems[sel])
        for cp in outcp:
            if cp is not None:
                cp.wait()

    return sc_gather


_sc_gather_cache = []


def _sc_gather(table, idx):
    if not _sc_gather_cache:
        _sc_gather_cache.append(_make_sc_gather())
    return _sc_gather_cache[0](table, idx)


def _dec_loss_body(q_ref, h_ref, x_ref, w1_ref, b1_ref, w2_ref, b2_ref,
                   mse_ref, com_ref):
    lvl = pl.program_id(0)
    bt = pl.program_id(1)

    @pl.when((lvl == 0) & (bt == 0))
    def _():
        mse_ref[...] = jnp.zeros((1, 1), jnp.float32)
        com_ref[...] = jnp.zeros((1, 1), jnp.float32)

    q = q_ref[0]
    d1 = jnp.maximum(
        jnp.dot(q, w1_ref[0], preferred_element_type=jnp.float32) + b1_ref[0],
        0.0)
    xh = jnp.dot(d1, w2_ref[0], preferred_element_type=jnp.float32) + b2_ref[0]
    diff = xh - x_ref[...]
    mse_ref[...] += jnp.sum(diff * diff) * (1.0 / (B * NF * LVL))

    @pl.when(lvl == LVL - 1)
    def _():
        dq = q - h_ref[0]
        com_ref[...] += jnp.sum(dq * dq) * (1.0 / (B * VQ * LVL))


def _dec_loss(q, h, x, dec_w1, dec_b1, dec_w2, dec_b2):
    return pl.pallas_call(
        _dec_loss_body,
        grid=(LVL, NB),
        in_specs=[
            pl.BlockSpec((1, BT, VQ), lambda l, b: (l, b, 0)),
            pl.BlockSpec((1, BT, VQ), lambda l, b: (l, b, 0)),
            pl.BlockSpec((BT, NF), lambda l, b: (b, l)),
            pl.BlockSpec((1, VQ, HID), lambda l, b: (l, 0, 0)),
            pl.BlockSpec((1, 1, HID), lambda l, b: (l, 0, 0)),
            pl.BlockSpec((1, HID, NF), lambda l, b: (l, 0, 0)),
            pl.BlockSpec((1, 1, NF), lambda l, b: (l, 0, 0)),
        ],
        out_specs=[
            pl.BlockSpec((1, 1), lambda l, b: (0, 0)),
            pl.BlockSpec((1, 1), lambda l, b: (0, 0)),
        ],
        out_shape=[
            jax.ShapeDtypeStruct((1, 1), jnp.float32),
            jax.ShapeDtypeStruct((1, 1), jnp.float32),
        ],
    )(q, h, x, dec_w1, dec_b1, dec_w2, dec_b2)


def kernel(x, enc_w1, enc_b1, enc_w2, enc_b2, dec_w1, dec_b1, dec_w2, dec_b2,
           codebook):
    eb1 = enc_b1.reshape(LVL, 1, HID)
    eb2 = enc_b2.reshape(LVL, 1, VQ)
    db1 = dec_b1.reshape(LVL, 1, HID)
    db2 = dec_b2.reshape(LVL, 1, NF)

    h, idx = _enc_argmin(x, enc_w1, eb1, enc_w2, eb2, codebook)
    q = _sc_gather(codebook.reshape(LVL * CB, VQ), idx.reshape(TOTB))
    mse, com = _dec_loss(q.reshape(LVL, B, VQ), h, x, dec_w1, db1, dec_w2, db2)
    return (mse.reshape(()), com.reshape(()))
